# baseline (speedup 1.0000x reference)
import functools

import jax
import jax.numpy as jnp
from jax import lax
from jax.experimental import pallas as pl
from jax.experimental.pallas import tpu as pltpu
from jax.experimental.pallas import tpu_sc as plsc

_NC = 2
_NS = 16
_NW = _NC * _NS
_CHUNK = 128
_DEPTH = 4


def _row_block(n):
  for b in (1000, 500, 256, 200, 128, 100, 64, 50, 40, 32, 25, 16, 8):
    if n % b == 0:
      return b
  return n


def _matmul48(x, wc):
  n, k = x.shape
  blk = _row_block(n)

  def body(x_ref, w_ref, tab_ref, r_ref):
    t = jnp.dot(x_ref[...], w_ref[...], preferred_element_type=jnp.float32)
    tab_ref[...] = t[:, 0:32]
    r_ref[...] = t[:, 32:48]

  return pl.pallas_call(
      body,
      grid=(n // blk,),
      in_specs=[
          pl.BlockSpec((blk, k), lambda i: (i, 0)),
          pl.BlockSpec((k, 48), lambda i: (0, 0)),
      ],
      out_specs=[
          pl.BlockSpec((blk, 32), lambda i: (i, 0)),
          pl.BlockSpec((blk, 16), lambda i: (i, 0)),
      ],
      out_shape=[
          jax.ShapeDtypeStruct((n, 32), jnp.float32),
          jax.ShapeDtypeStruct((n, 16), jnp.float32),
      ],
  )(x, wc)


def _edge_pass(tab, src3, dst3, u2, zero, np_rows, nch, wide):
  epw = nch * _CHUNK
  rps = np_rows // _NS
  npair = nch // 2
  w = 32 if wide else 16
  mesh = plsc.VectorSubcoreMesh(
      core_axis_name="c", subcore_axis_name="s",
      num_cores=_NC, num_subcores=_NS)

  @functools.partial(
      pl.kernel,
      out_type=jax.ShapeDtypeStruct((_NC, np_rows, w), jnp.float32),
      mesh=mesh,
      scratch_types=[
          pltpu.VMEM_SHARED((np_rows, w), jnp.float32),
          pltpu.VMEM((nch, _CHUNK), jnp.int32),
          pltpu.VMEM((nch, _CHUNK), jnp.int32),
          pltpu.VMEM((epw + 16,), jnp.float32),
          [pltpu.VMEM((_CHUNK, 32), jnp.float32) for _ in range(_DEPTH)],
          [pltpu.VMEM((_CHUNK, w), jnp.float32) for _ in range(_DEPTH)],
          [pltpu.SemaphoreType.DMA for _ in range(_DEPTH)],
          [pltpu.SemaphoreType.DMA for _ in range(_DEPTH)],
      ],
      compiler_params=pltpu.CompilerParams(use_tc_tiling_on_sc=False),
  )
  def k(tab_hbm, src_hbm, dst_hbm, u_hbm, zero_hbm, out_hbm,
        agg_sh, srcv, dstv, uv, rows, msg, gs, ss):
    c = lax.axis_index("c")
    s = lax.axis_index("s")
    wid = c * _NS + s
    scope = jax.named_scope

    sc_stage = scope("sc_stage")
    sc_stage.__enter__()
    st0 = pltpu.async_copy(src_hbm.at[wid], srcv, gs[0])
    st1 = pltpu.async_copy(dst_hbm.at[wid], dstv, gs[1])
    st2 = pltpu.async_copy(u_hbm.at[wid], uv.at[pl.ds(0, epw)], gs[2])
    st3 = pltpu.async_copy(zero_hbm.at[pl.ds(s * rps, rps)],
                           agg_sh.at[pl.ds(s * rps, rps)], gs[3])
    st2.wait()

    def clipv(t, _):
      v = uv[pl.ds(t * 16, 16)]
      uv[pl.ds(t * 16, 16)] = jnp.minimum(jnp.maximum(v, 0.0), 1.0)
      return 0

    lax.fori_loop(0, epw // 16, clipv, 0, unroll=4)

    if wide:
      lane = lax.iota(jnp.int32, 16)
      tailv = jnp.where(lane == 0, 1.0, 0.0).astype(jnp.float32)

      def init_row(e2, _):
        for q in range(_DEPTH):
          msg[q][e2, pl.ds(16, 16)] = tailv
        return 0

      lax.fori_loop(0, _CHUNK, init_row, 0)
    st0.wait()
    st1.wait()
    st3.wait()
    plsc.subcore_barrier()
    sc_stage.__exit__(None, None, None)
    sc_loop = scope("sc_loop")
    sc_loop.__enter__()

    def compute(i, rowsb, msgb):
      base = i * _CHUNK

      def group(g, _):
        u16 = uv[pl.ds(base + g * 16, 16)]
        for k2 in range(16):
          e2 = g * 16 + k2
          u = u16[k2]
          r0 = rowsb[e2, pl.ds(0, 16)]
          d = rowsb[e2, pl.ds(16, 16)]
          msgb[e2, pl.ds(0, 16)] = r0 + u * d
        return 0

      lax.fori_loop(0, _CHUNK // 16, group, 0, unroll=4)

    for q in range(_DEPTH - 1):
      pltpu.async_copy(tab_hbm.at[srcv.at[q]], rows[q], gs[q])

    def quad(j, _):
      for q in range(_DEPTH):
        i = _DEPTH * j + q
        pre = i + _DEPTH - 1
        qp = (q + _DEPTH - 1) % _DEPTH

        @pl.when(pre < nch)
        def _():
          pltpu.async_copy(tab_hbm.at[srcv.at[pre]], rows[qp], gs[qp])

        pltpu.make_async_copy(tab_hbm.at[srcv.at[i]], rows[q], gs[q]).wait()

        @pl.when(i >= _DEPTH)
        def _():
          pltpu.make_async_copy(msg[q], agg_sh.at[dstv.at[i]], ss[q]).wait()

        compute(i, rows[q], msg[q])
        pltpu.async_copy(msg[q], agg_sh.at[dstv.at[i]], ss[q], add=True)
      return 0

    lax.fori_loop(0, nch // _DEPTH, quad, 0)
    for q in range(_DEPTH):
      pltpu.make_async_copy(msg[q], agg_sh.at[dstv.at[0]], ss[q]).wait()
    sc_loop.__exit__(None, None, None)
    with scope("sc_out"):
      plsc.subcore_barrier()
      pltpu.sync_copy(agg_sh.at[pl.ds(s * rps, rps)],
                      out_hbm.at[c, pl.ds(s * rps, rps)])

  return k(tab, src3, dst3, u2, zero)


def _mid(part1, r1, b1, wc2, n):
  blk = _row_block(n)

  def body(p_ref, r1_ref, b1_ref, w_ref, x1_ref, tab2_ref, r2_ref, deg_ref):
    p = p_ref[...]
    agg = p[0] + p[1]
    degc = jnp.maximum(agg[:, 16:17], 1.0)
    h = agg[:, 0:16] / degc + r1_ref[...] + b1_ref[...]
    x1 = jnp.where(h > 0, h, jnp.exp(h) - 1.0)
    t2 = jnp.dot(x1, w_ref[...], preferred_element_type=jnp.float32)
    x1_ref[...] = x1
    tab2_ref[...] = t2[:, 0:32]
    r2_ref[...] = t2[:, 32:48]
    deg_ref[...] = degc

  return pl.pallas_call(
      body,
      grid=(n // blk,),
      in_specs=[
          pl.BlockSpec((2, blk, 32), lambda i: (0, i, 0)),
          pl.BlockSpec((blk, 16), lambda i: (i, 0)),
          pl.BlockSpec((1, 16), lambda i: (0, 0)),
          pl.BlockSpec((16, 48), lambda i: (0, 0)),
      ],
      out_specs=[
          pl.BlockSpec((blk, 16), lambda i: (i, 0)),
          pl.BlockSpec((blk, 32), lambda i: (i, 0)),
          pl.BlockSpec((blk, 16), lambda i: (i, 0)),
          pl.BlockSpec((blk, 1), lambda i: (i, 0)),
      ],
      out_shape=[
          jax.ShapeDtypeStruct((n, 16), jnp.float32),
          jax.ShapeDtypeStruct((n, 32), jnp.float32),
          jax.ShapeDtypeStruct((n, 16), jnp.float32),
          jax.ShapeDtypeStruct((n, 1), jnp.float32),
      ],
  )(part1, r1, b1, wc2)


def _fin(part2, r2, b2, deg, n):
  blk = _row_block(n)

  def body(p_ref, r2_ref, b2_ref, deg_ref, out_ref):
    p = p_ref[...]
    agg = p[0] + p[1]
    x2 = agg / deg_ref[...] + r2_ref[...] + b2_ref[...]
    m = jnp.max(x2, axis=1, keepdims=True)
    sh = x2 - m
    lse = jnp.log(jnp.sum(jnp.exp(sh), axis=1, keepdims=True))
    out_ref[...] = sh - lse

  return pl.pallas_call(
      body,
      grid=(n // blk,),
      in_specs=[
          pl.BlockSpec((2, blk, 16), lambda i: (0, i, 0)),
          pl.BlockSpec((blk, 16), lambda i: (i, 0)),
          pl.BlockSpec((1, 16), lambda i: (0, 0)),
          pl.BlockSpec((blk, 1), lambda i: (i, 0)),
      ],
      out_specs=pl.BlockSpec((blk, 16), lambda i: (i, 0)),
      out_shape=jax.ShapeDtypeStruct((n, 16), jnp.float32),
  )(part2, r2, b2, deg)


def kernel(x, edge_index, edge_attr, W1, root1, bias1, W2, root2, bias2):
  n, _ = x.shape
  e = edge_index.shape[1]
  hid = W1.shape[2]
  cls = W2.shape[2]

  src = edge_index[0]
  dst = edge_index[1]
  u = edge_attr[:, 0]

  per = _NW * _CHUNK * _DEPTH
  e_pad = ((e + per - 1) // per) * per
  nch = e_pad // (_NW * _CHUNK)
  np_rows = ((n + 1 + _NS * 8 - 1) // (_NS * 8)) * (_NS * 8)

  srcp = jnp.pad(src, (0, e_pad - e)).reshape(_NW, nch, _CHUNK)
  dstp = jnp.pad(dst, (0, e_pad - e),
                 constant_values=n).reshape(_NW, nch, _CHUNK)
  up = jnp.pad(u, (0, e_pad - e)).reshape(_NW, nch * _CHUNK)
  zero32 = jnp.zeros((np_rows, 32), jnp.float32)
  zero16 = jnp.zeros((np_rows, 16), jnp.float32)

  wc1 = jnp.concatenate([W1[0], W1[1] - W1[0], root1], axis=1)
  wc2 = jnp.concatenate([W2[0], W2[1] - W2[0], root2], axis=1)
  b1 = bias1.reshape(1, hid)
  b2 = bias2.reshape(1, cls)

  tab1, r1 = _matmul48(x, wc1)
  part1 = _edge_pass(tab1, srcp, dstp, up, zero32, np_rows, nch, wide=True)
  x1, tab2, r2, deg = _mid(part1, r1, b1, wc2, n)
  part2 = _edge_pass(tab2, srcp, dstp, up, zero16, np_rows, nch, wide=False)
  out = _fin(part2, r2, b2, deg, n)
  return out, x1


# speedup vs baseline: 1.3939x; 1.3939x over previous
import functools

import jax
import jax.numpy as jnp
from jax import lax
from jax.experimental import pallas as pl
from jax.experimental.pallas import tpu as pltpu
from jax.experimental.pallas import tpu_sc as plsc

_NC = 2
_NS = 16
_NW = _NC * _NS
_CHUNK = 128
_DEPTH = 4


def _row_block(n):
  for b in (1000, 500, 256, 200, 128, 100, 64, 50, 40, 32, 25, 16, 8):
    if n % b == 0:
      return b
  return n


def _matmul48(x, wc):
  n, k = x.shape
  blk = _row_block(n)

  def body(x_ref, w_ref, tab_ref, r_ref):
    t = jnp.dot(x_ref[...], w_ref[...], preferred_element_type=jnp.float32)
    tab_ref[...] = t[:, 0:32]
    r_ref[...] = t[:, 32:48]

  return pl.pallas_call(
      body,
      grid=(n // blk,),
      in_specs=[
          pl.BlockSpec((blk, k), lambda i: (i, 0)),
          pl.BlockSpec((k, 48), lambda i: (0, 0)),
      ],
      out_specs=[
          pl.BlockSpec((blk, 32), lambda i: (i, 0)),
          pl.BlockSpec((blk, 16), lambda i: (i, 0)),
      ],
      out_shape=[
          jax.ShapeDtypeStruct((n, 32), jnp.float32),
          jax.ShapeDtypeStruct((n, 16), jnp.float32),
      ],
  )(x, wc)


def _edge_pass(tab, src3, dst3, u2, zero, np_rows, nch, wide):
  epw = nch * _CHUNK
  rps = np_rows // _NS
  npair = nch // 2
  w = 32 if wide else 16
  mesh = plsc.VectorSubcoreMesh(
      core_axis_name="c", subcore_axis_name="s",
      num_cores=_NC, num_subcores=_NS)

  @functools.partial(
      pl.kernel,
      out_type=jax.ShapeDtypeStruct((_NC, np_rows, w), jnp.float32),
      mesh=mesh,
      scratch_types=[
          pltpu.VMEM_SHARED((np_rows, w), jnp.float32),
          pltpu.VMEM((nch, _CHUNK), jnp.int32),
          pltpu.VMEM((nch, _CHUNK), jnp.int32),
          pltpu.VMEM((epw + 16,), jnp.float32),
          [pltpu.VMEM((_CHUNK, 32), jnp.float32) for _ in range(_DEPTH)],
          [pltpu.VMEM((_CHUNK, w), jnp.float32) for _ in range(_DEPTH)],
          [pltpu.SemaphoreType.DMA for _ in range(_DEPTH)],
          [pltpu.SemaphoreType.DMA for _ in range(_DEPTH)],
      ],
      compiler_params=pltpu.CompilerParams(use_tc_tiling_on_sc=False),
  )
  def k(tab_hbm, src_hbm, dst_hbm, u_hbm, zero_hbm, out_hbm,
        agg_sh, srcv, dstv, uv, rows, msg, gs, ss):
    c = lax.axis_index("c")
    s = lax.axis_index("s")
    wid = c * _NS + s
    scope = jax.named_scope

    sc_stage = scope("sc_stage")
    sc_stage.__enter__()
    st0 = pltpu.async_copy(src_hbm.at[wid], srcv, gs[0])
    st1 = pltpu.async_copy(dst_hbm.at[wid], dstv, gs[1])
    st2 = pltpu.async_copy(u_hbm.at[wid], uv.at[pl.ds(0, epw)], gs[2])
    st3 = pltpu.async_copy(zero_hbm.at[pl.ds(s * rps, rps)],
                           agg_sh.at[pl.ds(s * rps, rps)], gs[3])
    st2.wait()

    def clipv(t, _):
      v = uv[pl.ds(t * 16, 16)]
      uv[pl.ds(t * 16, 16)] = jnp.minimum(jnp.maximum(v, 0.0), 1.0)
      return 0

    lax.fori_loop(0, epw // 16, clipv, 0, unroll=4)

    if wide:
      lane = lax.iota(jnp.int32, 16)
      tailv = jnp.where(lane == 0, 1.0, 0.0).astype(jnp.float32)

      def init_row(e2, _):
        for q in range(_DEPTH):
          msg[q][e2, pl.ds(16, 16)] = tailv
        return 0

      lax.fori_loop(0, _CHUNK, init_row, 0)
    st0.wait()
    st1.wait()
    st3.wait()
    plsc.subcore_barrier()
    sc_stage.__exit__(None, None, None)
    sc_loop = scope("sc_loop")
    sc_loop.__enter__()

    def compute(i, rowsb, msgb):
      base = i * _CHUNK

      def group(g, _):
        u16 = uv[pl.ds(base + g * 16, 16)]
        for k2 in range(16):
          e2 = g * 16 + k2
          u = u16[k2]
          r0 = rowsb[e2, pl.ds(0, 16)]
          d = rowsb[e2, pl.ds(16, 16)]
          msgb[e2, pl.ds(0, 16)] = r0 + u * d
        return 0

      lax.fori_loop(0, _CHUNK // 16, group, 0, unroll=4)

    for q in range(_DEPTH - 1):
      pltpu.async_copy(tab_hbm.at[srcv.at[q]], rows[q], gs[q])

    def quad(j, _):
      for q in range(_DEPTH):
        i = _DEPTH * j + q
        pre = i + _DEPTH - 1
        qp = (q + _DEPTH - 1) % _DEPTH

        @pl.when(pre < nch)
        def _():
          pltpu.async_copy(tab_hbm.at[srcv.at[pre]], rows[qp], gs[qp])

        pltpu.make_async_copy(tab_hbm.at[srcv.at[i]], rows[q], gs[q]).wait()

        @pl.when(i >= _DEPTH)
        def _():
          pltpu.make_async_copy(msg[q], agg_sh.at[dstv.at[i]], ss[q]).wait()

        compute(i, rows[q], msg[q])
        pltpu.async_copy(msg[q], agg_sh.at[dstv.at[i]], ss[q], add=True)
      return 0

    lax.fori_loop(0, nch // _DEPTH, quad, 0)
    for q in range(_DEPTH):
      pltpu.make_async_copy(msg[q], agg_sh.at[dstv.at[0]], ss[q]).wait()
    sc_loop.__exit__(None, None, None)
    with scope("sc_out"):
      plsc.subcore_barrier()
      pltpu.sync_copy(agg_sh.at[pl.ds(s * rps, rps)],
                      out_hbm.at[c, pl.ds(s * rps, rps)])

  return k(tab, src3, dst3, u2, zero)


def _mid(part1, r1, b1, wc2, n):
  blk = _row_block(n)

  def body(p_ref, r1_ref, b1_ref, w_ref, x1_ref, tab2_ref, r2_ref, deg_ref):
    p = p_ref[...]
    agg = p[0] + p[1]
    degc = jnp.maximum(agg[:, 16:17], 1.0)
    h = agg[:, 0:16] / degc + r1_ref[...] + b1_ref[...]
    x1 = jnp.where(h > 0, h, jnp.exp(h) - 1.0)
    t2 = jnp.dot(x1, w_ref[...], preferred_element_type=jnp.float32)
    x1_ref[...] = x1
    tab2_ref[...] = t2[:, 0:32]
    r2_ref[...] = t2[:, 32:48]
    deg_ref[...] = degc

  return pl.pallas_call(
      body,
      grid=(n // blk,),
      in_specs=[
          pl.BlockSpec((2, blk, 32), lambda i: (0, i, 0)),
          pl.BlockSpec((blk, 16), lambda i: (i, 0)),
          pl.BlockSpec((1, 16), lambda i: (0, 0)),
          pl.BlockSpec((16, 48), lambda i: (0, 0)),
      ],
      out_specs=[
          pl.BlockSpec((blk, 16), lambda i: (i, 0)),
          pl.BlockSpec((blk, 32), lambda i: (i, 0)),
          pl.BlockSpec((blk, 16), lambda i: (i, 0)),
          pl.BlockSpec((blk, 1), lambda i: (i, 0)),
      ],
      out_shape=[
          jax.ShapeDtypeStruct((n, 16), jnp.float32),
          jax.ShapeDtypeStruct((n, 32), jnp.float32),
          jax.ShapeDtypeStruct((n, 16), jnp.float32),
          jax.ShapeDtypeStruct((n, 1), jnp.float32),
      ],
  )(part1, r1, b1, wc2)


def _fin(part2, r2, b2, deg, n):
  blk = _row_block(n)

  def body(p_ref, r2_ref, b2_ref, deg_ref, out_ref):
    p = p_ref[...]
    agg = p[0] + p[1]
    x2 = agg / deg_ref[...] + r2_ref[...] + b2_ref[...]
    m = jnp.max(x2, axis=1, keepdims=True)
    sh = x2 - m
    lse = jnp.log(jnp.sum(jnp.exp(sh), axis=1, keepdims=True))
    out_ref[...] = sh - lse

  return pl.pallas_call(
      body,
      grid=(n // blk,),
      in_specs=[
          pl.BlockSpec((2, blk, 16), lambda i: (0, i, 0)),
          pl.BlockSpec((blk, 16), lambda i: (i, 0)),
          pl.BlockSpec((1, 16), lambda i: (0, 0)),
          pl.BlockSpec((blk, 1), lambda i: (i, 0)),
      ],
      out_specs=pl.BlockSpec((blk, 16), lambda i: (i, 0)),
      out_shape=jax.ShapeDtypeStruct((n, 16), jnp.float32),
  )(part2, r2, b2, deg)


def kernel(x, edge_index, edge_attr, W1, root1, bias1, W2, root2, bias2):
  n, _ = x.shape
  e = edge_index.shape[1]
  hid = W1.shape[2]
  cls = W2.shape[2]

  src = edge_index[0]
  dst = edge_index[1]
  u = edge_attr[:, 0]

  per = _NW * _CHUNK * _DEPTH
  e_pad = ((e + per - 1) // per) * per
  nch = e_pad // (_NW * _CHUNK)
  np_rows = ((n + 1 + _NS * 8 - 1) // (_NS * 8)) * (_NS * 8)

  pad_idx = jnp.arange(e_pad - e, dtype=jnp.int32)
  srcp = jnp.concatenate([src, pad_idx % n]).reshape(_NW, nch, _CHUNK)
  dstp = jnp.concatenate(
      [dst, n + pad_idx % (np_rows - n)]).reshape(_NW, nch, _CHUNK)
  up = jnp.pad(u, (0, e_pad - e)).reshape(_NW, nch * _CHUNK)
  zero32 = jnp.zeros((np_rows, 32), jnp.float32)
  zero16 = jnp.zeros((np_rows, 16), jnp.float32)

  wc1 = jnp.concatenate([W1[0], W1[1] - W1[0], root1], axis=1)
  wc2 = jnp.concatenate([W2[0], W2[1] - W2[0], root2], axis=1)
  b1 = bias1.reshape(1, hid)
  b2 = bias2.reshape(1, cls)

  tab1, r1 = _matmul48(x, wc1)
  part1 = _edge_pass(tab1, srcp, dstp, up, zero32, np_rows, nch, wide=True)
  x1, tab2, r2, deg = _mid(part1, r1, b1, wc2, n)
  part2 = _edge_pass(tab2, srcp, dstp, up, zero16, np_rows, nch, wide=False)
  out = _fin(part2, r2, b2, deg, n)
  return out, x1


# speedup vs baseline: 1.6429x; 1.1786x over previous
import functools

import jax
import jax.numpy as jnp
from jax import lax
from jax.experimental import pallas as pl
from jax.experimental.pallas import tpu as pltpu
from jax.experimental.pallas import tpu_sc as plsc

_NC = 2
_NS = 16
_NW = _NC * _NS
_CHUNK = 128
_DEPTH = 4


def _row_block(n):
  for b in (1000, 500, 256, 200, 128, 100, 64, 50, 40, 32, 25, 16, 8):
    if n % b == 0:
      return b
  return n


def _matmul48(x, wc):
  n, k = x.shape
  blk = _row_block(n)

  def body(x_ref, w_ref, tab_ref, r_ref):
    t = jnp.dot(x_ref[...], w_ref[...], preferred_element_type=jnp.float32)
    tab_ref[...] = t[:, 0:32]
    r_ref[...] = t[:, 32:48]

  return pl.pallas_call(
      body,
      grid=(n // blk,),
      in_specs=[
          pl.BlockSpec((blk, k), lambda i: (i, 0)),
          pl.BlockSpec((k, 48), lambda i: (0, 0)),
      ],
      out_specs=[
          pl.BlockSpec((blk, 32), lambda i: (i, 0)),
          pl.BlockSpec((blk, 16), lambda i: (i, 0)),
      ],
      out_shape=[
          jax.ShapeDtypeStruct((n, 32), jnp.float32),
          jax.ShapeDtypeStruct((n, 16), jnp.float32),
      ],
  )(x, wc)


def _edge_pass(tab, src3, dst3, u2, zero, np_rows, nch, wide):
  epw = nch * _CHUNK
  rps = np_rows // _NS
  npair = nch // 2
  w = 32 if wide else 16
  mesh = plsc.VectorSubcoreMesh(
      core_axis_name="c", subcore_axis_name="s",
      num_cores=_NC, num_subcores=_NS)

  @functools.partial(
      pl.kernel,
      out_type=jax.ShapeDtypeStruct((_NC, np_rows, w), jnp.float32),
      mesh=mesh,
      scratch_types=[
          pltpu.VMEM_SHARED((np_rows, w), jnp.float32),
          pltpu.VMEM((nch, _CHUNK), jnp.int32),
          pltpu.VMEM((nch, _CHUNK), jnp.int32),
          pltpu.VMEM((epw + 16,), jnp.float32),
          [pltpu.VMEM((_CHUNK, 32), jnp.float32) for _ in range(_DEPTH)],
          [pltpu.VMEM((_CHUNK, w), jnp.float32) for _ in range(_DEPTH)],
          [pltpu.SemaphoreType.DMA for _ in range(_DEPTH)],
          [pltpu.SemaphoreType.DMA for _ in range(_DEPTH)],
      ],
      compiler_params=pltpu.CompilerParams(use_tc_tiling_on_sc=False),
  )
  def k(tab_hbm, src_hbm, dst_hbm, u_hbm, zero_hbm, out_hbm,
        agg_sh, srcv, dstv, uv, rows, msg, gs, ss):
    c = lax.axis_index("c")
    s = lax.axis_index("s")
    wid = c * _NS + s
    scope = jax.named_scope

    sc_stage = scope("sc_stage")
    sc_stage.__enter__()
    st0 = pltpu.async_copy(src_hbm.at[wid], srcv, gs[0])
    st1 = pltpu.async_copy(dst_hbm.at[wid], dstv, gs[1])
    st2 = pltpu.async_copy(u_hbm.at[wid], uv.at[pl.ds(0, epw)], gs[2])
    st3 = pltpu.async_copy(zero_hbm.at[pl.ds(s * rps, rps)],
                           agg_sh.at[pl.ds(s * rps, rps)], gs[3])
    st2.wait()

    def clipv(t, _):
      v = uv[pl.ds(t * 16, 16)]
      uv[pl.ds(t * 16, 16)] = jnp.minimum(jnp.maximum(v, 0.0), 1.0)
      return 0

    lax.fori_loop(0, epw // 16, clipv, 0, unroll=4)

    if wide:
      lane = lax.iota(jnp.int32, 16)
      tailv = jnp.where(lane == 0, 1.0, 0.0).astype(jnp.float32)

      def init_row(e2, _):
        for q in range(_DEPTH):
          msg[q][e2, pl.ds(16, 16)] = tailv
        return 0

      lax.fori_loop(0, _CHUNK, init_row, 0)
    st0.wait()
    st1.wait()
    st3.wait()
    plsc.subcore_barrier()
    sc_stage.__exit__(None, None, None)
    sc_loop = scope("sc_loop")
    sc_loop.__enter__()

    def compute(i, rowsb, msgb):
      base = i * _CHUNK

      def group(g, _):
        u16 = uv[pl.ds(base + g * 16, 16)]
        for k2 in range(16):
          e2 = g * 16 + k2
          u = u16[k2]
          r0 = rowsb[e2, pl.ds(0, 16)]
          d = rowsb[e2, pl.ds(16, 16)]
          msgb[e2, pl.ds(0, 16)] = r0 + u * d
        return 0

      lax.fori_loop(0, _CHUNK // 16, group, 0, unroll=4)

    for q in range(_DEPTH - 1):
      pltpu.async_copy(tab_hbm.at[srcv.at[q]], rows[q], gs[q])

    def quad(j, _):
      for q in range(_DEPTH):
        i = _DEPTH * j + q
        pre = i + _DEPTH - 1
        qp = (q + _DEPTH - 1) % _DEPTH

        @pl.when(pre < nch)
        def _():
          pltpu.async_copy(tab_hbm.at[srcv.at[pre]], rows[qp], gs[qp])

        pltpu.make_async_copy(tab_hbm.at[srcv.at[i]], rows[q], gs[q]).wait()

        @pl.when(i >= _DEPTH)
        def _():
          pltpu.make_async_copy(msg[q], agg_sh.at[dstv.at[i]], ss[q]).wait()

        compute(i, rows[q], msg[q])
        pltpu.async_copy(msg[q], agg_sh.at[dstv.at[i]], ss[q], add=True)
      return 0

    lax.fori_loop(0, nch // _DEPTH, quad, 0)
    for q in range(_DEPTH):
      pltpu.make_async_copy(msg[q], agg_sh.at[dstv.at[0]], ss[q]).wait()
    sc_loop.__exit__(None, None, None)
    with scope("sc_out"):
      plsc.subcore_barrier()
      pltpu.sync_copy(agg_sh.at[pl.ds(s * rps, rps)],
                      out_hbm.at[c, pl.ds(s * rps, rps)])

  return k(tab, src3, dst3, u2, zero)


def _mid(part1, r1, b1, wc2, n):
  blk = _row_block(n)

  def body(p_ref, r1_ref, b1_ref, w_ref, x1_ref, tab2_ref, r2_ref, deg_ref):
    p = p_ref[...]
    agg = p[0] + p[1]
    degc = jnp.maximum(agg[:, 16:17], 1.0)
    h = agg[:, 0:16] / degc + r1_ref[...] + b1_ref[...]
    x1 = jnp.where(h > 0, h, jnp.exp(h) - 1.0)
    t2 = jnp.dot(x1, w_ref[...], preferred_element_type=jnp.float32)
    x1_ref[...] = x1
    tab2_ref[...] = t2[:, 0:32]
    r2_ref[...] = t2[:, 32:48]
    deg_ref[...] = degc

  return pl.pallas_call(
      body,
      grid=(n // blk,),
      in_specs=[
          pl.BlockSpec((2, blk, 32), lambda i: (0, i, 0)),
          pl.BlockSpec((blk, 16), lambda i: (i, 0)),
          pl.BlockSpec((1, 16), lambda i: (0, 0)),
          pl.BlockSpec((16, 48), lambda i: (0, 0)),
      ],
      out_specs=[
          pl.BlockSpec((blk, 16), lambda i: (i, 0)),
          pl.BlockSpec((blk, 32), lambda i: (i, 0)),
          pl.BlockSpec((blk, 16), lambda i: (i, 0)),
          pl.BlockSpec((blk, 1), lambda i: (i, 0)),
      ],
      out_shape=[
          jax.ShapeDtypeStruct((n, 16), jnp.float32),
          jax.ShapeDtypeStruct((n, 32), jnp.float32),
          jax.ShapeDtypeStruct((n, 16), jnp.float32),
          jax.ShapeDtypeStruct((n, 1), jnp.float32),
      ],
  )(part1, r1, b1, wc2)


def _fin(part2, r2, b2, deg, n):
  blk = _row_block(n)

  def body(p_ref, r2_ref, b2_ref, deg_ref, out_ref):
    p = p_ref[...]
    agg = p[0] + p[1]
    x2 = agg / deg_ref[...] + r2_ref[...] + b2_ref[...]
    m = jnp.max(x2, axis=1, keepdims=True)
    sh = x2 - m
    lse = jnp.log(jnp.sum(jnp.exp(sh), axis=1, keepdims=True))
    out_ref[...] = sh - lse

  return pl.pallas_call(
      body,
      grid=(n // blk,),
      in_specs=[
          pl.BlockSpec((2, blk, 16), lambda i: (0, i, 0)),
          pl.BlockSpec((blk, 16), lambda i: (i, 0)),
          pl.BlockSpec((1, 16), lambda i: (0, 0)),
          pl.BlockSpec((blk, 1), lambda i: (i, 0)),
      ],
      out_specs=pl.BlockSpec((blk, 16), lambda i: (i, 0)),
      out_shape=jax.ShapeDtypeStruct((n, 16), jnp.float32),
  )(part2, r2, b2, deg)


def kernel(x, edge_index, edge_attr, W1, root1, bias1, W2, root2, bias2):
  n, _ = x.shape
  e = edge_index.shape[1]
  hid = W1.shape[2]
  cls = W2.shape[2]

  src = edge_index[0]
  dst = edge_index[1]
  u = edge_attr[:, 0]

  per = _NW * _CHUNK * _DEPTH
  e_pad = ((e + per - 1) // per) * per
  nch = e_pad // (_NW * _CHUNK)
  np_rows = ((n + 1 + _NS * 8 - 1) // (_NS * 8)) * (_NS * 8)

  pad_idx = jnp.arange(e_pad - e, dtype=jnp.int32)
  srcp = jnp.concatenate([src, pad_idx % n]).reshape(_NW, nch, _CHUNK)
  dstp = jnp.concatenate(
      [dst, n + pad_idx % (np_rows - n)]).reshape(_NW, nch, _CHUNK)
  up = jnp.pad(u, (0, e_pad - e)).reshape(_NW, nch * _CHUNK)
  zero32 = jnp.zeros((np_rows, 32), jnp.float32)
  zero16 = jnp.zeros((np_rows, 16), jnp.float32)

  wc1 = jnp.concatenate([W1[0], W1[1] - W1[0], root1], axis=1)
  wc2 = jnp.concatenate([W2[0], W2[1] - W2[0], root2], axis=1)
  b1 = bias1.reshape(1, hid)
  b2 = bias2.reshape(1, cls)

  tab1, r1 = _matmul48(x, wc1)
  part1 = _edge_pass(tab1, srcp, dstp, up, zero32, np_rows, nch, wide=True)
  x1, tab2, r2, deg = _mid(part1, r1, b1, wc2, n)
  part2 = _edge_pass(tab2, srcp, dstp, up, zero32, np_rows, nch, wide=True)
  part2 = part2[:, :, 0:16]
  out = _fin(part2, r2, b2, deg, n)
  return out, x1


# speedup vs baseline: 1.6443x; 1.0009x over previous
import functools

import jax
import jax.numpy as jnp
from jax import lax
from jax.experimental import pallas as pl
from jax.experimental.pallas import tpu as pltpu
from jax.experimental.pallas import tpu_sc as plsc

_NC = 2
_NS = 16
_NW = _NC * _NS
_CHUNK = 128
_DEPTH = 4


def _row_block(n):
  for b in (1000, 500, 256, 200, 128, 100, 64, 50, 40, 32, 25, 16, 8):
    if n % b == 0:
      return b
  return n


def _matmul48(x, wc):
  n, k = x.shape
  blk = _row_block(n)

  def body(x_ref, w_ref, tab_ref, r_ref):
    t = jnp.dot(x_ref[...], w_ref[...], preferred_element_type=jnp.float32)
    tab_ref[...] = t[:, 0:32]
    r_ref[...] = t[:, 32:48]

  return pl.pallas_call(
      body,
      grid=(n // blk,),
      in_specs=[
          pl.BlockSpec((blk, k), lambda i: (i, 0)),
          pl.BlockSpec((k, 48), lambda i: (0, 0)),
      ],
      out_specs=[
          pl.BlockSpec((blk, 32), lambda i: (i, 0)),
          pl.BlockSpec((blk, 16), lambda i: (i, 0)),
      ],
      out_shape=[
          jax.ShapeDtypeStruct((n, 32), jnp.float32),
          jax.ShapeDtypeStruct((n, 16), jnp.float32),
      ],
  )(x, wc)


def _edge_pass(tab, src3, dst3, u2, zero, np_rows, nch, wide):
  epw = nch * _CHUNK
  rps = np_rows // _NS
  npair = nch // 2
  w = 32 if wide else 16
  mesh = plsc.VectorSubcoreMesh(
      core_axis_name="c", subcore_axis_name="s",
      num_cores=_NC, num_subcores=_NS)

  @functools.partial(
      pl.kernel,
      out_type=jax.ShapeDtypeStruct((_NC, np_rows, w), jnp.float32),
      mesh=mesh,
      scratch_types=[
          pltpu.VMEM_SHARED((np_rows, w), jnp.float32),
          pltpu.VMEM((nch, _CHUNK), jnp.int32),
          pltpu.VMEM((nch, _CHUNK), jnp.int32),
          pltpu.VMEM((epw + 16,), jnp.float32),
          [pltpu.VMEM((_CHUNK, 32), jnp.float32) for _ in range(_DEPTH)],
          [pltpu.VMEM((_CHUNK, w), jnp.float32) for _ in range(_DEPTH)],
          [pltpu.SemaphoreType.DMA for _ in range(_DEPTH)],
          [pltpu.SemaphoreType.DMA for _ in range(_DEPTH)],
      ],
      compiler_params=pltpu.CompilerParams(use_tc_tiling_on_sc=False),
  )
  def k(tab_hbm, src_hbm, dst_hbm, u_hbm, zero_hbm, out_hbm,
        agg_sh, srcv, dstv, uv, rows, msg, gs, ss):
    c = lax.axis_index("c")
    s = lax.axis_index("s")
    wid = c * _NS + s
    scope = jax.named_scope

    sc_stage = scope("sc_stage")
    sc_stage.__enter__()
    st0 = pltpu.async_copy(src_hbm.at[wid], srcv, gs[0])
    st1 = pltpu.async_copy(dst_hbm.at[wid], dstv, gs[1])
    st2 = pltpu.async_copy(u_hbm.at[wid], uv.at[pl.ds(0, epw)], gs[2])
    st3 = pltpu.async_copy(zero_hbm.at[pl.ds(s * rps, rps)],
                           agg_sh.at[pl.ds(s * rps, rps)], gs[3])
    st2.wait()

    def clipv(t, _):
      v = uv[pl.ds(t * 16, 16)]
      uv[pl.ds(t * 16, 16)] = jnp.minimum(jnp.maximum(v, 0.0), 1.0)
      return 0

    lax.fori_loop(0, epw // 16, clipv, 0, unroll=4)

    if wide:
      lane = lax.iota(jnp.int32, 16)
      tailv = jnp.where(lane == 0, 1.0, 0.0).astype(jnp.float32)

      def init_row(e2, _):
        for q in range(_DEPTH):
          msg[q][e2, pl.ds(16, 16)] = tailv
        return 0

      lax.fori_loop(0, _CHUNK, init_row, 0)
    st0.wait()
    st1.wait()
    st3.wait()
    plsc.subcore_barrier()
    sc_stage.__exit__(None, None, None)
    sc_loop = scope("sc_loop")
    sc_loop.__enter__()

    def compute(i, rowsb, msgb):
      base = i * _CHUNK

      def group(g, _):
        u16 = uv[pl.ds(base + g * 16, 16)]
        for k2 in range(16):
          e2 = g * 16 + k2
          u = u16[k2]
          r0 = rowsb[e2, pl.ds(0, 16)]
          d = rowsb[e2, pl.ds(16, 16)]
          msgb[e2, pl.ds(0, 16)] = r0 + u * d
        return 0

      lax.fori_loop(0, _CHUNK // 16, group, 0, unroll=4)

    for q in range(_DEPTH - 1):
      pltpu.async_copy(tab_hbm.at[srcv.at[q]], rows[q], gs[q])

    def quad(j, _):
      for q in range(_DEPTH):
        i = _DEPTH * j + q
        pre = i + _DEPTH - 1
        qp = (q + _DEPTH - 1) % _DEPTH

        @pl.when(pre < nch)
        def _():
          pltpu.async_copy(tab_hbm.at[srcv.at[pre]], rows[qp], gs[qp])

        pltpu.make_async_copy(tab_hbm.at[srcv.at[i]], rows[q], gs[q]).wait()

        @pl.when(i >= _DEPTH)
        def _():
          pltpu.make_async_copy(msg[q], agg_sh.at[dstv.at[i]], ss[q]).wait()

        compute(i, rows[q], msg[q])
        pltpu.async_copy(msg[q], agg_sh.at[dstv.at[i]], ss[q], add=True)
      return 0

    lax.fori_loop(0, nch // _DEPTH, quad, 0)
    for q in range(_DEPTH):
      pltpu.make_async_copy(msg[q], agg_sh.at[dstv.at[0]], ss[q]).wait()
    sc_loop.__exit__(None, None, None)
    with scope("sc_out"):
      plsc.subcore_barrier()
      pltpu.sync_copy(agg_sh.at[pl.ds(s * rps, rps)],
                      out_hbm.at[c, pl.ds(s * rps, rps)])

  return k(tab, src3, dst3, u2, zero)


def _mid(part1, r1, b1, wc2, n):
  blk = _row_block(n)

  def body(p_ref, r1_ref, b1_ref, w_ref, x1_ref, tab2_ref, r2_ref, deg_ref):
    p = p_ref[...]
    agg = p[0] + p[1]
    degc = jnp.maximum(agg[:, 16:17], 1.0)
    h = agg[:, 0:16] / degc + r1_ref[...] + b1_ref[...]
    x1 = jnp.where(h > 0, h, jnp.exp(h) - 1.0)
    t2 = jnp.dot(x1, w_ref[...], preferred_element_type=jnp.float32)
    x1_ref[...] = x1
    tab2_ref[...] = t2[:, 0:32]
    r2_ref[...] = t2[:, 32:48]
    deg_ref[...] = degc

  return pl.pallas_call(
      body,
      grid=(n // blk,),
      in_specs=[
          pl.BlockSpec((2, blk, 32), lambda i: (0, i, 0)),
          pl.BlockSpec((blk, 16), lambda i: (i, 0)),
          pl.BlockSpec((1, 16), lambda i: (0, 0)),
          pl.BlockSpec((16, 48), lambda i: (0, 0)),
      ],
      out_specs=[
          pl.BlockSpec((blk, 16), lambda i: (i, 0)),
          pl.BlockSpec((blk, 32), lambda i: (i, 0)),
          pl.BlockSpec((blk, 16), lambda i: (i, 0)),
          pl.BlockSpec((blk, 1), lambda i: (i, 0)),
      ],
      out_shape=[
          jax.ShapeDtypeStruct((n, 16), jnp.float32),
          jax.ShapeDtypeStruct((n, 32), jnp.float32),
          jax.ShapeDtypeStruct((n, 16), jnp.float32),
          jax.ShapeDtypeStruct((n, 1), jnp.float32),
      ],
  )(part1, r1, b1, wc2)


def _fin(part2, r2, b2, deg, n):
  blk = _row_block(n)

  def body(p_ref, r2_ref, b2_ref, deg_ref, out_ref):
    p = p_ref[...]
    agg = p[0, :, 0:16] + p[1, :, 0:16]
    x2 = agg / deg_ref[...] + r2_ref[...] + b2_ref[...]
    m = jnp.max(x2, axis=1, keepdims=True)
    sh = x2 - m
    lse = jnp.log(jnp.sum(jnp.exp(sh), axis=1, keepdims=True))
    out_ref[...] = sh - lse

  return pl.pallas_call(
      body,
      grid=(n // blk,),
      in_specs=[
          pl.BlockSpec((2, blk, 32), lambda i: (0, i, 0)),
          pl.BlockSpec((blk, 16), lambda i: (i, 0)),
          pl.BlockSpec((1, 16), lambda i: (0, 0)),
          pl.BlockSpec((blk, 1), lambda i: (i, 0)),
      ],
      out_specs=pl.BlockSpec((blk, 16), lambda i: (i, 0)),
      out_shape=jax.ShapeDtypeStruct((n, 16), jnp.float32),
  )(part2, r2, b2, deg)


def kernel(x, edge_index, edge_attr, W1, root1, bias1, W2, root2, bias2):
  n, _ = x.shape
  e = edge_index.shape[1]
  hid = W1.shape[2]
  cls = W2.shape[2]

  src = edge_index[0]
  dst = edge_index[1]
  u = edge_attr[:, 0]

  per = _NW * _CHUNK * _DEPTH
  e_pad = ((e + per - 1) // per) * per
  nch = e_pad // (_NW * _CHUNK)
  np_rows = ((n + 1 + _NS * 8 - 1) // (_NS * 8)) * (_NS * 8)

  pad_idx = jnp.arange(e_pad - e, dtype=jnp.int32)
  srcp = jnp.concatenate([src, pad_idx % n]).reshape(_NW, nch, _CHUNK)
  dstp = jnp.concatenate(
      [dst, n + pad_idx % (np_rows - n)]).reshape(_NW, nch, _CHUNK)
  up = jnp.pad(u, (0, e_pad - e)).reshape(_NW, nch * _CHUNK)
  zero32 = jnp.zeros((np_rows, 32), jnp.float32)

  wc1 = jnp.concatenate([W1[0], W1[1] - W1[0], root1], axis=1)
  wc2 = jnp.concatenate([W2[0], W2[1] - W2[0], root2], axis=1)
  b1 = bias1.reshape(1, hid)
  b2 = bias2.reshape(1, cls)

  tab1, r1 = _matmul48(x, wc1)
  part1 = _edge_pass(tab1, srcp, dstp, up, zero32, np_rows, nch, wide=True)
  x1, tab2, r2, deg = _mid(part1, r1, b1, wc2, n)
  part2 = _edge_pass(tab2, srcp, dstp, up, zero32, np_rows, nch, wide=True)
  out = _fin(part2, r2, b2, deg, n)
  return out, x1
